# baseline (speedup 1.0000x reference)
import functools

import jax
import jax.numpy as jnp
from jax import lax
from jax.experimental import pallas as pl
from jax.experimental.pallas import tpu as pltpu
from jax.experimental.pallas import tpu_sc as plsc

MEM_N = 65536
B_N = 256
U_N = 128
DECAY_C = 0.95
TILE = 1024
NTILES = MEM_N // TILE
LANES = 16


def _lstm_body(x_ref, hprev_ref, cst_ref, wk_ref, wr_ref, b_ref,
               h_ref, c_ref, nh_ref):
    z = (jnp.dot(x_ref[...], wk_ref[...],
                 preferred_element_type=jnp.float32,
                 precision=lax.Precision.HIGHEST)
         + jnp.dot(hprev_ref[...], wr_ref[...],
                   preferred_element_type=jnp.float32,
                   precision=lax.Precision.HIGHEST)
         + b_ref[...])
    zi = z[:, :U_N]
    zf = z[:, U_N:2 * U_N]
    zc = z[:, 2 * U_N:3 * U_N]
    zo = z[:, 3 * U_N:]
    i_g = jax.nn.sigmoid(zi)
    f_g = jax.nn.sigmoid(zf)
    c_new = f_g * cst_ref[...] + i_g * jnp.tanh(zc)
    o_g = jax.nn.sigmoid(zo)
    h = o_g * jnp.tanh(c_new)
    h_ref[...] = h
    c_ref[...] = c_new
    n = jnp.sqrt(jnp.maximum(jnp.sum(h * h, axis=1, keepdims=True), 1e-12))
    nh_ref[...] = h / n


def _lstm_call(x, h_tm1, cst_tm1, Wk, Wr, b):
    f32 = jnp.float32
    return pl.pallas_call(
        _lstm_body,
        out_shape=[
            jax.ShapeDtypeStruct((B_N, U_N), f32),
            jax.ShapeDtypeStruct((B_N, U_N), f32),
            jax.ShapeDtypeStruct((B_N, U_N), f32),
        ],
    )(x, h_tm1, cst_tm1, Wk, Wr, b.reshape(1, 4 * U_N))


def _main_body(wg_ref, m_ref, cwu_ref, cwlu_ref, cwr_ref, nh_ref, h_ref,
               cww_o, cwr_o, cwu_o, cwlu_o, mem_o, read_o,
               arg1_o, arg2_o, scale_o,
               rm_s, r1_s, r2_s):
    i = pl.program_id(0)
    wg = jax.nn.sigmoid(wg_ref[0, 0])
    c_ww = wg * cwr_ref[...] + (1.0 - wg) + cwlu_ref[...]
    cww_o[...] = c_ww

    m = m_ref[...]
    n = jnp.sqrt(jnp.maximum(jnp.sum(m * m, axis=1, keepdims=True), 1e-12))
    n_m = m / n
    cos = lax.dot_general(n_m, nh_ref[...], (((1,), (1,)), ((), ())),
                          preferred_element_type=jnp.float32,
                          precision=lax.Precision.HIGHEST)
    x_max = jnp.max(cos, axis=1, keepdims=True)
    unnorm = jnp.exp(cos - x_max)
    c_wr = unnorm / jnp.sum(unnorm, axis=1, keepdims=True)
    cwr_o[...] = c_wr

    c_wu = DECAY_C * cwu_ref[...] + c_wr + c_ww
    cwu_o[...] = c_wu
    cwlu_o[...] = jnp.zeros_like(c_wu)

    mem_o[...] = 256.0 * m + jnp.dot(c_ww, h_ref[...],
                                     preferred_element_type=jnp.float32)

    contrib = lax.dot_general(c_wr, m, (((0,), (0,)), ((), ())),
                              preferred_element_type=jnp.float32)

    tm = jnp.min(c_wu, axis=0, keepdims=True)
    gid = lax.broadcasted_iota(jnp.int32, (TILE, B_N), 0).astype(jnp.float32)
    m1 = jnp.where(c_wu == tm, gid, -1.0)
    t1l = jnp.max(m1, axis=0, keepdims=True)
    m2 = jnp.where(m1 == t1l, -1.0, m1)
    t2l = jnp.max(m2, axis=0, keepdims=True)
    base = (i * TILE).astype(jnp.float32)
    t1 = t1l + base
    t2 = jnp.where(t2l >= 0.0, t2l + base, -1.0)

    @pl.when(i == 0)
    def _():
        read_o[...] = contrib
        rm_s[...] = tm
        r1_s[...] = t1
        r2_s[...] = t2

    @pl.when(i > 0)
    def _():
        read_o[...] += contrib
        rm = rm_s[...]
        r1 = r1_s[...]
        r2 = r2_s[...]
        strictly = tm < rm
        equal = tm == rm
        rm_s[...] = jnp.where(strictly, tm, rm)
        r1_s[...] = jnp.where(strictly | equal, t1, r1)
        r2_s[...] = jnp.where(strictly, t2,
                              jnp.where(equal, jnp.maximum(r1, t2), r2))

    @pl.when(i == NTILES - 1)
    def _():
        a1 = r1_s[...].astype(jnp.int32)
        arg1_o[...] = a1
        arg2_o[...] = r2_s[...].astype(jnp.int32)
        rows = lax.broadcasted_iota(jnp.int32, (B_N, B_N), 0)
        oh_hi = (rows == (a1 >> 8)).astype(jnp.float32)
        oh_lo = (rows == (a1 & 255)).astype(jnp.float32)
        eq_hi = lax.dot_general(oh_hi, oh_hi, (((0,), (0,)), ((), ())),
                                preferred_element_type=jnp.float32,
                                precision=lax.Precision.HIGHEST)
        eq_lo = lax.dot_general(oh_lo, oh_lo, (((0,), (0,)), ((), ())),
                                preferred_element_type=jnp.float32,
                                precision=lax.Precision.HIGHEST)
        eq = eq_hi * eq_lo
        scale_o[...] = jnp.sum(eq, axis=0, keepdims=True)


def _main_call(write_gate, m_tm1, c_wu_tm1, c_wlu_tm1, c_wr_tm1, n_h, h):
    f32 = jnp.float32
    i32 = jnp.int32
    big = lambda: pl.BlockSpec((TILE, B_N), lambda i: (i, 0))
    full = lambda r, c: pl.BlockSpec((r, c), lambda i: (0, 0))
    return pl.pallas_call(
        _main_body,
        grid=(NTILES,),
        in_specs=[
            pl.BlockSpec(memory_space=pltpu.SMEM),
            pl.BlockSpec((TILE, U_N), lambda i: (i, 0)),
            big(),
            big(),
            big(),
            full(B_N, U_N),
            full(B_N, U_N),
        ],
        out_specs=[
            big(),
            big(),
            big(),
            big(),
            pl.BlockSpec((TILE, U_N), lambda i: (i, 0)),
            full(B_N, U_N),
            full(1, B_N),
            full(1, B_N),
            full(1, B_N),
        ],
        out_shape=[
            jax.ShapeDtypeStruct((MEM_N, B_N), f32),
            jax.ShapeDtypeStruct((MEM_N, B_N), f32),
            jax.ShapeDtypeStruct((MEM_N, B_N), f32),
            jax.ShapeDtypeStruct((MEM_N, B_N), f32),
            jax.ShapeDtypeStruct((MEM_N, U_N), f32),
            jax.ShapeDtypeStruct((B_N, U_N), f32),
            jax.ShapeDtypeStruct((1, B_N), i32),
            jax.ShapeDtypeStruct((1, B_N), i32),
            jax.ShapeDtypeStruct((1, B_N), f32),
        ],
        scratch_shapes=[
            pltpu.VMEM((1, B_N), f32),
            pltpu.VMEM((1, B_N), f32),
            pltpu.VMEM((1, B_N), f32),
        ],
        compiler_params=pltpu.CompilerParams(
            dimension_semantics=("arbitrary",)),
    )(write_gate, m_tm1, c_wu_tm1, c_wlu_tm1, c_wr_tm1, n_h, h)


@functools.cache
def _get_sc_fix():
    mesh = plsc.VectorSubcoreMesh(core_axis_name="c", subcore_axis_name="s")
    cp = pltpu.CompilerParams()
    if "needs_layout_passes" in pltpu.CompilerParams.__dataclass_fields__:
        import dataclasses
        cp = dataclasses.replace(cp, needs_layout_passes=False)
    return functools.partial(
        pl.kernel,
        out_type=(),
        mesh=mesh,
        compiler_params=cp,
        scratch_types=[
            pltpu.VMEM((LANES,), jnp.int32),
            pltpu.VMEM((LANES,), jnp.int32),
            pltpu.VMEM((LANES,), jnp.float32),
            pltpu.VMEM((2 * LANES, LANES), jnp.float32),
            pltpu.VMEM((LANES, U_N), jnp.float32),
            pltpu.VMEM((LANES, U_N), jnp.float32),
            pltpu.SemaphoreType.DMA,
        ],
    )(_sc_fix_body)


def _sc_fix_body(arg1_hbm, arg2_hbm, scale_hbm, m_hbm, mem_ref, clwu_ref,
                 a1_vm, a2_vm, sc_vm, chunks_vm, rows_vm, mrows_vm, sem):
    cid = lax.axis_index("c")
    sid = lax.axis_index("s")
    base = sid * LANES
    pltpu.sync_copy(arg1_hbm.at[pl.ds(base, LANES)], a1_vm)
    pltpu.sync_copy(arg2_hbm.at[pl.ds(base, LANES)], a2_vm)
    pltpu.sync_copy(scale_hbm.at[pl.ds(base, LANES)], sc_vm)
    band1 = a1_vm[...]
    band2 = a2_vm[...]
    scb = sc_vm[...]
    lane = lax.iota(jnp.int32, LANES)

    def sel_i32(vec, k):
        return jnp.max(jnp.where(lane == k, vec, jnp.int32(-2147483648)))

    @pl.when(cid == 0)
    def _():
        rset = [sel_i32(band1, k) for k in range(LANES)]
        band2f = jnp.where(band2 >= 0, band2, band1)
        rset += [sel_i32(band2f, k) for k in range(LANES)]
        gets = [pltpu.async_copy(clwu_ref.at[r, pl.ds(base, LANES)],
                                 chunks_vm.at[j], sem)
                for j, r in enumerate(rset)]
        for g in gets:
            g.wait()
        for j, r in enumerate(rset):
            mask = (band1 == r) | (band2 == r)
            chunks_vm[j] = jnp.where(mask, 1.0, chunks_vm[j])
        puts = [pltpu.async_copy(chunks_vm.at[j],
                                 clwu_ref.at[r, pl.ds(base, LANES)], sem)
                for j, r in enumerate(rset)]
        for p in puts:
            p.wait()

    @pl.when(cid == 1)
    def _():
        rset = [sel_i32(band1, k) for k in range(LANES)]
        gets = [pltpu.async_copy(mem_ref.at[r], rows_vm.at[k], sem)
                for k, r in enumerate(rset)]
        gets += [pltpu.async_copy(m_hbm.at[r], mrows_vm.at[k], sem)
                 for k, r in enumerate(rset)]
        for g in gets:
            g.wait()
        plsc.subcore_barrier()
        for k in range(LANES):
            sc = jnp.max(jnp.where(lane == k, scb, -jnp.inf))
            for c2 in range(U_N // LANES):
                sl = (k, pl.ds(c2 * LANES, LANES))
                rows_vm[sl] = rows_vm[sl] - sc * mrows_vm[sl]
        puts = [pltpu.async_copy(rows_vm.at[k], mem_ref.at[r], sem)
                for k, r in enumerate(rset)]
        for p in puts:
            p.wait()


def kernel(inputs, r_tm1, m_tm1, c_wu_tm1, c_wlu_tm1, c_wr_tm1, c_ww_tm1,
           h_tm1, cst_tm1, Wk, Wr, b, write_gate):
    del c_ww_tm1
    x = jnp.concatenate([inputs, r_tm1], axis=-1)
    h, c_new, n_h = _lstm_call(x, h_tm1, cst_tm1, Wk, Wr, b)

    (c_ww, c_wr, c_wu, c_wlu0, mem_pre, read, arg1, arg2, scale) = _main_call(
        write_gate.reshape(1, 1).astype(jnp.float32),
        m_tm1, c_wu_tm1, c_wlu_tm1, c_wr_tm1, n_h, h)

    mem_ref = jax.new_ref(mem_pre)
    clwu_ref = jax.new_ref(c_wlu0)
    _get_sc_fix()(arg1.reshape(B_N), arg2.reshape(B_N), scale.reshape(B_N),
                  m_tm1, mem_ref, clwu_ref)
    memory = mem_ref[...]
    c_wlu = clwu_ref[...]

    return (read, read, memory, c_wu, c_wlu, c_wr, c_ww, h, c_new)


# speedup vs baseline: 1.1272x; 1.1272x over previous
import functools

import jax
import jax.numpy as jnp
from jax import lax
from jax.experimental import pallas as pl
from jax.experimental.pallas import tpu as pltpu
from jax.experimental.pallas import tpu_sc as plsc

MEM_N = 65536
B_N = 256
U_N = 128
DECAY_C = 0.95
TILE = 2048
NTILES = MEM_N // TILE
LANES = 16


def _lstm_body(x_ref, hprev_ref, cst_ref, wk_ref, wr_ref, b_ref,
               h_ref, c_ref, nh_ref):
    z = (jnp.dot(x_ref[...], wk_ref[...],
                 preferred_element_type=jnp.float32,
                 precision=lax.Precision.HIGHEST)
         + jnp.dot(hprev_ref[...], wr_ref[...],
                   preferred_element_type=jnp.float32,
                   precision=lax.Precision.HIGHEST)
         + b_ref[...])
    zi = z[:, :U_N]
    zf = z[:, U_N:2 * U_N]
    zc = z[:, 2 * U_N:3 * U_N]
    zo = z[:, 3 * U_N:]
    i_g = jax.nn.sigmoid(zi)
    f_g = jax.nn.sigmoid(zf)
    c_new = f_g * cst_ref[...] + i_g * jnp.tanh(zc)
    o_g = jax.nn.sigmoid(zo)
    h = o_g * jnp.tanh(c_new)
    h_ref[...] = h
    c_ref[...] = c_new
    n = jnp.sqrt(jnp.maximum(jnp.sum(h * h, axis=1, keepdims=True), 1e-12))
    nh_ref[...] = h / n


def _lstm_call(x, h_tm1, cst_tm1, Wk, Wr, b):
    f32 = jnp.float32
    return pl.pallas_call(
        _lstm_body,
        out_shape=[
            jax.ShapeDtypeStruct((B_N, U_N), f32),
            jax.ShapeDtypeStruct((B_N, U_N), f32),
            jax.ShapeDtypeStruct((B_N, U_N), f32),
        ],
    )(x, h_tm1, cst_tm1, Wk, Wr, b.reshape(1, 4 * U_N))


def _main_body(wg_ref, m_ref, cwu_ref, cwlu_ref, cwr_ref, nh_ref, h_ref,
               cww_o, cwr_o, cwu_o, cwlu_o, mem_o, read_o,
               arg1_o, arg2_o, scale_o,
               rm_s, r1_s, r2_s):
    i = pl.program_id(0)
    wg = jax.nn.sigmoid(wg_ref[0, 0])
    c_ww = wg * cwr_ref[...] + (1.0 - wg) + cwlu_ref[...]
    cww_o[...] = c_ww

    m = m_ref[...]
    n = jnp.sqrt(jnp.maximum(jnp.sum(m * m, axis=1, keepdims=True), 1e-12))
    n_m = m / n
    dn = (((1,), (1,)), ((), ()))
    nm_hi = n_m.astype(jnp.bfloat16)
    nm_lo = (n_m - nm_hi.astype(jnp.float32)).astype(jnp.bfloat16)
    nh = nh_ref[...]
    nh_hi = nh.astype(jnp.bfloat16)
    nh_lo = (nh - nh_hi.astype(jnp.float32)).astype(jnp.bfloat16)
    cos = (lax.dot_general(nm_hi, nh_hi, dn,
                           preferred_element_type=jnp.float32)
           + (lax.dot_general(nm_hi, nh_lo, dn,
                              preferred_element_type=jnp.float32)
              + lax.dot_general(nm_lo, nh_hi, dn,
                                preferred_element_type=jnp.float32)))
    unnorm = jnp.exp(cos)
    c_wr = unnorm / jnp.sum(unnorm, axis=1, keepdims=True)
    cwr_o[...] = c_wr

    c_wu = DECAY_C * cwu_ref[...] + c_wr + c_ww
    cwu_o[...] = c_wu
    cwlu_o[...] = jnp.zeros_like(c_wu)

    mem_o[...] = 256.0 * m + jnp.dot(c_ww, h_ref[...],
                                     preferred_element_type=jnp.float32)

    contrib = lax.dot_general(c_wr, m, (((0,), (0,)), ((), ())),
                              preferred_element_type=jnp.float32)

    tm = jnp.min(c_wu, axis=0, keepdims=True)
    gid = lax.broadcasted_iota(jnp.int32, (TILE, B_N), 0).astype(jnp.float32)
    m1 = jnp.where(c_wu == tm, gid, -1.0)
    t1l = jnp.max(m1, axis=0, keepdims=True)
    m2 = jnp.where(m1 == t1l, -1.0, m1)
    t2l = jnp.max(m2, axis=0, keepdims=True)
    base = (i * TILE).astype(jnp.float32)
    t1 = t1l + base
    t2 = jnp.where(t2l >= 0.0, t2l + base, -1.0)

    @pl.when(i == 0)
    def _():
        read_o[...] = contrib
        rm_s[...] = tm
        r1_s[...] = t1
        r2_s[...] = t2

    @pl.when(i > 0)
    def _():
        read_o[...] += contrib
        rm = rm_s[...]
        r1 = r1_s[...]
        r2 = r2_s[...]
        strictly = tm < rm
        equal = tm == rm
        rm_s[...] = jnp.where(strictly, tm, rm)
        r1_s[...] = jnp.where(strictly | equal, t1, r1)
        r2_s[...] = jnp.where(strictly, t2,
                              jnp.where(equal, jnp.maximum(r1, t2), r2))

    @pl.when(i == NTILES - 1)
    def _():
        a1 = r1_s[...].astype(jnp.int32)
        arg1_o[...] = a1
        arg2_o[...] = r2_s[...].astype(jnp.int32)
        rows = lax.broadcasted_iota(jnp.int32, (B_N, B_N), 0)
        oh_hi = (rows == (a1 >> 8)).astype(jnp.float32)
        oh_lo = (rows == (a1 & 255)).astype(jnp.float32)
        eq_hi = lax.dot_general(oh_hi, oh_hi, (((0,), (0,)), ((), ())),
                                preferred_element_type=jnp.float32,
                                precision=lax.Precision.HIGHEST)
        eq_lo = lax.dot_general(oh_lo, oh_lo, (((0,), (0,)), ((), ())),
                                preferred_element_type=jnp.float32,
                                precision=lax.Precision.HIGHEST)
        eq = eq_hi * eq_lo
        scale_o[...] = jnp.sum(eq, axis=0, keepdims=True)


def _main_call(write_gate, m_tm1, c_wu_tm1, c_wlu_tm1, c_wr_tm1, n_h, h):
    f32 = jnp.float32
    i32 = jnp.int32
    big = lambda: pl.BlockSpec((TILE, B_N), lambda i: (i, 0))
    full = lambda r, c: pl.BlockSpec((r, c), lambda i: (0, 0))
    return pl.pallas_call(
        _main_body,
        grid=(NTILES,),
        in_specs=[
            pl.BlockSpec(memory_space=pltpu.SMEM),
            pl.BlockSpec((TILE, U_N), lambda i: (i, 0)),
            big(),
            big(),
            big(),
            full(B_N, U_N),
            full(B_N, U_N),
        ],
        out_specs=[
            big(),
            big(),
            big(),
            big(),
            pl.BlockSpec((TILE, U_N), lambda i: (i, 0)),
            full(B_N, U_N),
            full(1, B_N),
            full(1, B_N),
            full(1, B_N),
        ],
        out_shape=[
            jax.ShapeDtypeStruct((MEM_N, B_N), f32),
            jax.ShapeDtypeStruct((MEM_N, B_N), f32),
            jax.ShapeDtypeStruct((MEM_N, B_N), f32),
            jax.ShapeDtypeStruct((MEM_N, B_N), f32),
            jax.ShapeDtypeStruct((MEM_N, U_N), f32),
            jax.ShapeDtypeStruct((B_N, U_N), f32),
            jax.ShapeDtypeStruct((1, B_N), i32),
            jax.ShapeDtypeStruct((1, B_N), i32),
            jax.ShapeDtypeStruct((1, B_N), f32),
        ],
        scratch_shapes=[
            pltpu.VMEM((1, B_N), f32),
            pltpu.VMEM((1, B_N), f32),
            pltpu.VMEM((1, B_N), f32),
        ],
        compiler_params=pltpu.CompilerParams(
            dimension_semantics=("arbitrary",)),
    )(write_gate, m_tm1, c_wu_tm1, c_wlu_tm1, c_wr_tm1, n_h, h)


@functools.cache
def _get_sc_fix():
    mesh = plsc.VectorSubcoreMesh(core_axis_name="c", subcore_axis_name="s")
    cp = pltpu.CompilerParams()
    if "needs_layout_passes" in pltpu.CompilerParams.__dataclass_fields__:
        import dataclasses
        cp = dataclasses.replace(cp, needs_layout_passes=False)
    return functools.partial(
        pl.kernel,
        out_type=(),
        mesh=mesh,
        compiler_params=cp,
        scratch_types=[
            pltpu.VMEM((LANES,), jnp.int32),
            pltpu.VMEM((LANES,), jnp.int32),
            pltpu.VMEM((LANES,), jnp.float32),
            pltpu.VMEM((2 * LANES, LANES), jnp.float32),
            pltpu.VMEM((LANES, U_N), jnp.float32),
            pltpu.VMEM((LANES, U_N), jnp.float32),
            pltpu.SemaphoreType.DMA,
        ],
    )(_sc_fix_body)


def _sc_fix_body(arg1_hbm, arg2_hbm, scale_hbm, m_hbm, mem_ref, clwu_ref,
                 a1_vm, a2_vm, sc_vm, chunks_vm, rows_vm, mrows_vm, sem):
    cid = lax.axis_index("c")
    sid = lax.axis_index("s")
    base = sid * LANES
    pltpu.sync_copy(arg1_hbm.at[pl.ds(base, LANES)], a1_vm)
    pltpu.sync_copy(arg2_hbm.at[pl.ds(base, LANES)], a2_vm)
    pltpu.sync_copy(scale_hbm.at[pl.ds(base, LANES)], sc_vm)
    band1 = a1_vm[...]
    band2 = a2_vm[...]
    scb = sc_vm[...]
    lane = lax.iota(jnp.int32, LANES)

    def sel_i32(vec, k):
        return jnp.max(jnp.where(lane == k, vec, jnp.int32(-2147483648)))

    @pl.when(cid == 0)
    def _():
        rset = [sel_i32(band1, k) for k in range(LANES)]
        band2f = jnp.where(band2 >= 0, band2, band1)
        rset += [sel_i32(band2f, k) for k in range(LANES)]
        gets = [pltpu.async_copy(clwu_ref.at[r, pl.ds(base, LANES)],
                                 chunks_vm.at[j], sem)
                for j, r in enumerate(rset)]
        for g in gets:
            g.wait()
        for j, r in enumerate(rset):
            mask = (band1 == r) | (band2 == r)
            chunks_vm[j] = jnp.where(mask, 1.0, chunks_vm[j])
        puts = [pltpu.async_copy(chunks_vm.at[j],
                                 clwu_ref.at[r, pl.ds(base, LANES)], sem)
                for j, r in enumerate(rset)]
        for p in puts:
            p.wait()

    @pl.when(cid == 1)
    def _():
        rset = [sel_i32(band1, k) for k in range(LANES)]
        gets = [pltpu.async_copy(mem_ref.at[r], rows_vm.at[k], sem)
                for k, r in enumerate(rset)]
        gets += [pltpu.async_copy(m_hbm.at[r], mrows_vm.at[k], sem)
                 for k, r in enumerate(rset)]
        for g in gets:
            g.wait()
        plsc.subcore_barrier()
        for k in range(LANES):
            sc = jnp.max(jnp.where(lane == k, scb, -jnp.inf))
            for c2 in range(U_N // LANES):
                sl = (k, pl.ds(c2 * LANES, LANES))
                rows_vm[sl] = rows_vm[sl] - sc * mrows_vm[sl]
        puts = [pltpu.async_copy(rows_vm.at[k], mem_ref.at[r], sem)
                for k, r in enumerate(rset)]
        for p in puts:
            p.wait()


def kernel(inputs, r_tm1, m_tm1, c_wu_tm1, c_wlu_tm1, c_wr_tm1, c_ww_tm1,
           h_tm1, cst_tm1, Wk, Wr, b, write_gate):
    del c_ww_tm1
    x = jnp.concatenate([inputs, r_tm1], axis=-1)
    h, c_new, n_h = _lstm_call(x, h_tm1, cst_tm1, Wk, Wr, b)

    (c_ww, c_wr, c_wu, c_wlu0, mem_pre, read, arg1, arg2, scale) = _main_call(
        write_gate.reshape(1, 1).astype(jnp.float32),
        m_tm1, c_wu_tm1, c_wlu_tm1, c_wr_tm1, n_h, h)

    mem_ref = jax.new_ref(mem_pre)
    clwu_ref = jax.new_ref(c_wlu0)
    _get_sc_fix()(arg1.reshape(B_N), arg2.reshape(B_N), scale.reshape(B_N),
                  m_tm1, mem_ref, clwu_ref)
    memory = mem_ref[...]
    c_wlu = clwu_ref[...]

    return (read, read, memory, c_wu, c_wlu, c_wr, c_ww, h, c_new)
